# baseline (speedup 1.0000x reference)
import functools

import jax
import jax.numpy as jnp
from jax.experimental import pallas as pl
from jax.experimental.pallas import tpu as pltpu

_N = 20000
_B = 512
_ROWS = 2048


def _iou_kernel(b1_ref, b2t_ref, ious_ref, amax_ref):
    b1 = b1_ref[...]
    b2 = b2t_ref[...]

    ones_row = jnp.ones((1, _B), jnp.float32)

    def bc(v):
        return jax.lax.dot_general(
            v, ones_row, (((1,), (0,)), ((), ())),
            preferred_element_type=jnp.float32,
        )

    im_a = bc(b1[:, 0:1])
    x1a = bc(b1[:, 1:2])
    y1a = bc(b1[:, 2:3])
    x2a = bc(b1[:, 3:4])
    y2a = bc(b1[:, 4:5])
    area_a = bc(b1[:, 5:6])

    im_b = b2[0:1, :]
    x1b = b2[1:2, :]
    y1b = b2[2:3, :]
    x2b = b2[3:4, :]
    y2b = b2[4:5, :]
    area_b = b2[5:6, :]

    iw = jnp.maximum(jnp.minimum(x2a, x2b) - jnp.maximum(x1a, x1b), 0.0)
    ih = jnp.maximum(jnp.minimum(y2a, y2b) - jnp.maximum(y1a, y1b), 0.0)
    inter = iw * ih
    iou = inter / ((area_a + area_b) - inter)
    iou = jnp.where(im_a != im_b, 0.0, iou)
    ious_ref[...] = iou

    mx = jnp.max(iou, axis=1, keepdims=True)
    col = jax.lax.broadcasted_iota(jnp.int32, iou.shape, 1)
    amax_ref[...] = jnp.min(
        jnp.where(iou == mx, col, _B), axis=1, keepdims=True
    )


def _pack(boxes):
    im = boxes[:, 0:1]
    x1 = boxes[:, 1:2]
    y1 = boxes[:, 2:3]
    x2 = boxes[:, 3:4]
    y2 = boxes[:, 4:5]
    area = (x2 - x1 + 1.0) * (y2 - y1 + 1.0)
    return jnp.concatenate([im, x1, y1, x2 + 1.0, y2 + 1.0, area], axis=1)


@functools.partial(jax.jit, static_argnames=())
def kernel(boxes1, boxes2):
    b1p = _pack(boxes1)
    b2p = _pack(boxes2).T
    grid = (pl.cdiv(_N, _ROWS),)
    ious, amax = pl.pallas_call(
        _iou_kernel,
        grid=grid,
        in_specs=[
            pl.BlockSpec((_ROWS, 6), lambda i: (i, 0)),
            pl.BlockSpec((6, _B), lambda i: (0, 0)),
        ],
        out_specs=[
            pl.BlockSpec((_ROWS, _B), lambda i: (i, 0)),
            pl.BlockSpec((_ROWS, 1), lambda i: (i, 0)),
        ],
        out_shape=[
            jax.ShapeDtypeStruct((_N, _B), jnp.float32),
            jax.ShapeDtypeStruct((_N, 1), jnp.int32),
        ],
        compiler_params=pltpu.CompilerParams(
            dimension_semantics=("parallel",),
        ),
    )(b1p, b2p)
    return amax.reshape(_N), ious


# speedup vs baseline: 1.0864x; 1.0864x over previous
import functools

import jax
import jax.numpy as jnp
from jax.experimental import pallas as pl
from jax.experimental.pallas import tpu as pltpu

_N = 20000
_B = 512
_ROWS = 2048


def _iou_kernel(b1_ref, b2t_ref, ious_ref, amax_ref):
    b1 = b1_ref[...]
    b2 = b2t_ref[...]

    im_a = b1[:, 0:1]
    x1a = b1[:, 1:2]
    y1a = b1[:, 2:3]
    x2a = b1[:, 3:4]
    y2a = b1[:, 4:5]
    area_a = b1[:, 5:6]

    im_b = b2[0:1, :]
    x1b = b2[1:2, :]
    y1b = b2[2:3, :]
    x2b = b2[3:4, :]
    y2b = b2[4:5, :]
    area_b = b2[5:6, :]

    iw = jnp.maximum(jnp.minimum(x2a, x2b) - jnp.maximum(x1a, x1b), 0.0)
    ih = jnp.maximum(jnp.minimum(y2a, y2b) - jnp.maximum(y1a, y1b), 0.0)
    inter = iw * ih
    iou = inter / ((area_a + area_b) - inter)
    iou = jnp.where(im_a != im_b, 0.0, iou)
    ious_ref[...] = iou

    v0, v1, v2, v3 = (iou[:, i * 128:(i + 1) * 128] for i in range(4))
    m01 = jnp.maximum(v0, v1)
    g01 = (v1 > v0).astype(jnp.int32)
    m23 = jnp.maximum(v2, v3)
    g23 = jnp.where(v3 > v2, 3, 2)
    m = jnp.maximum(m01, m23)
    gg = jnp.where(m23 > m01, g23, g01)
    lane = jax.lax.broadcasted_iota(jnp.int32, m.shape, 1)
    colc = gg * 128 + lane
    mx = jnp.max(m, axis=1, keepdims=True)
    amax_ref[...] = jnp.min(
        jnp.where(m == mx, colc, _B), axis=1, keepdims=True
    )


def _pack(boxes):
    im = boxes[:, 0:1]
    x1 = boxes[:, 1:2]
    y1 = boxes[:, 2:3]
    x2 = boxes[:, 3:4]
    y2 = boxes[:, 4:5]
    area = (x2 - x1 + 1.0) * (y2 - y1 + 1.0)
    return jnp.concatenate([im, x1, y1, x2 + 1.0, y2 + 1.0, area], axis=1)


@functools.partial(jax.jit, static_argnames=())
def kernel(boxes1, boxes2):
    b1p = _pack(boxes1)
    b2p = _pack(boxes2).T
    grid = (pl.cdiv(_N, _ROWS),)
    ious, amax = pl.pallas_call(
        _iou_kernel,
        grid=grid,
        in_specs=[
            pl.BlockSpec((_ROWS, 6), lambda i: (i, 0)),
            pl.BlockSpec((6, _B), lambda i: (0, 0)),
        ],
        out_specs=[
            pl.BlockSpec((_ROWS, _B), lambda i: (i, 0)),
            pl.BlockSpec((_ROWS, 1), lambda i: (i, 0)),
        ],
        out_shape=[
            jax.ShapeDtypeStruct((_N, _B), jnp.float32),
            jax.ShapeDtypeStruct((_N, 1), jnp.int32),
        ],
        compiler_params=pltpu.CompilerParams(
            dimension_semantics=("parallel",),
        ),
    )(b1p, b2p)
    return amax.reshape(_N), ious
